# baseline (speedup 1.0000x reference)
import functools

import jax
import jax.numpy as jnp
from jax import lax
from jax.experimental import pallas as pl
from jax.experimental.pallas import tpu as pltpu
from jax.experimental.pallas import tpu_sc as plsc

NC = 2
NS = 16
NW = NC * NS
IPG = 128


@functools.lru_cache(maxsize=None)
def _make_gather(B, V, D, C):
    assert B % (NW * C) == 0 and C % IPG == 0
    b_per_w = B // NW
    n_chunks = b_per_w // C
    K = C // IPG
    mesh = plsc.VectorSubcoreMesh(core_axis_name="c", subcore_axis_name="s")

    @functools.partial(
        pl.kernel,
        out_type=jax.ShapeDtypeStruct((B, D), jnp.float32),
        mesh=mesh,
        compiler_params=pltpu.CompilerParams(use_tc_tiling_on_sc=False),
        scratch_types=[
            pltpu.VMEM((K, IPG), jnp.int32),
            pltpu.VMEM((C, D), jnp.float32),
            pltpu.SemaphoreType.DMA,
        ],
    )
    def grab(idx_hbm, table_hbm, out_hbm, idx_v, rows_v, sem):
        wid = lax.axis_index("s") * NC + lax.axis_index("c")
        base = wid * b_per_w

        def body(i, carry):
            off = pl.multiple_of(base + i * C, C)
            pltpu.sync_copy(idx_hbm.at[pl.ds(pl.multiple_of(off // IPG, K), K)], idx_v)
            copies = []
            for j in range(K):
                copies.append(pltpu.async_copy(
                    table_hbm.at[idx_v.at[j]],
                    rows_v.at[pl.ds(j * IPG, IPG)],
                    sem,
                ))
            for c in copies:
                c.wait()
            pltpu.sync_copy(rows_v, out_hbm.at[pl.ds(off, C)])
            return carry

        lax.fori_loop(0, n_chunks, body, 0)

    return grab


def kernel(input, table):
    BATCH, SEQ = input.shape
    V, D = table.shape
    B = BATCH * SEQ
    C = 1024
    idx = input.reshape(B // IPG, IPG).astype(jnp.int32)
    out = _make_gather(B, V, D, C)(idx, table)
    return out.reshape(BATCH, SEQ, D)


# speedup vs baseline: 1.0283x; 1.0283x over previous
import functools

import jax
import jax.numpy as jnp
from jax import lax
from jax.experimental import pallas as pl
from jax.experimental.pallas import tpu as pltpu
from jax.experimental.pallas import tpu_sc as plsc

NC = 2
NS = 16
NW = NC * NS
IPG = 128


@functools.lru_cache(maxsize=None)
def _make_gather(B, V, D, C, NBUF):
    b_per_w = B // NW
    n_chunks = b_per_w // C
    n_g = n_chunks // NBUF
    K = C // IPG
    assert B % (NW * C) == 0 and C % IPG == 0 and n_chunks % NBUF == 0
    mesh = plsc.VectorSubcoreMesh(core_axis_name="c", subcore_axis_name="s")

    @functools.partial(
        pl.kernel,
        out_type=jax.ShapeDtypeStruct((B, D), jnp.float32),
        mesh=mesh,
        compiler_params=pltpu.CompilerParams(use_tc_tiling_on_sc=False),
        scratch_types=[
            pltpu.VMEM((NBUF, C), jnp.int32),
            pltpu.VMEM((NBUF, C, D), jnp.float32),
            pltpu.SemaphoreType.DMA((NBUF,)),
            pltpu.SemaphoreType.DMA((NBUF,)),
            pltpu.SemaphoreType.DMA,
        ],
    )
    def grab(idx_hbm, table_hbm, out_hbm, idx_v, rows_v, sem_i, sem_o, sem_g):
        wid = lax.axis_index("s") * NC + lax.axis_index("c")
        base = wid * b_per_w

        def chunk_off(i):
            return pl.multiple_of(base + i * C, C)

        for b in range(NBUF):
            pltpu.async_copy(
                idx_hbm.at[pl.ds(chunk_off(b), C)], idx_v.at[b], sem_i.at[b])

        def body(g, carry):
            for b in range(NBUF):
                i = g * NBUF + b
                off = chunk_off(i)

                @pl.when(g > 0)
                def _():
                    pltpu.make_async_copy(
                        rows_v.at[b],
                        out_hbm.at[pl.ds(chunk_off(i - NBUF), C)],
                        sem_o.at[b],
                    ).wait()

                pltpu.make_async_copy(
                    idx_hbm.at[pl.ds(off, C)], idx_v.at[b], sem_i.at[b]
                ).wait()

                copies = [
                    pltpu.async_copy(
                        table_hbm.at[idx_v.at[b, pl.ds(j * IPG, IPG)]],
                        rows_v.at[b, pl.ds(j * IPG, IPG)],
                        sem_g,
                    )
                    for j in range(K)
                ]
                for c in copies:
                    c.wait()

                @pl.when(g < n_g - 1)
                def _():
                    pltpu.async_copy(
                        idx_hbm.at[pl.ds(chunk_off(i + NBUF), C)],
                        idx_v.at[b],
                        sem_i.at[b],
                    )

                pltpu.async_copy(
                    rows_v.at[b], out_hbm.at[pl.ds(off, C)], sem_o.at[b])
            return carry

        lax.fori_loop(0, n_g, body, 0)

        for b in range(NBUF):
            i = n_chunks - NBUF + b
            pltpu.make_async_copy(
                rows_v.at[b],
                out_hbm.at[pl.ds(chunk_off(i), C)],
                sem_o.at[b],
            ).wait()

    return grab


def kernel(input, table):
    BATCH, SEQ = input.shape
    V, D = table.shape
    B = BATCH * SEQ
    idx = input.reshape(B).astype(jnp.int32)
    out = _make_gather(B, V, D, 512, 2)(idx, table)
    return out.reshape(BATCH, SEQ, D)


# speedup vs baseline: 1.0293x; 1.0010x over previous
import functools

import jax
import jax.numpy as jnp
from jax import lax
from jax.experimental import pallas as pl
from jax.experimental.pallas import tpu as pltpu
from jax.experimental.pallas import tpu_sc as plsc

NC = 2
NS = 16
NW = NC * NS
IPG = 128


@functools.lru_cache(maxsize=None)
def _make_gather(BATCH, SEQ, V, D, CR, NBUF):
    rows_per_w = BATCH // NW
    n_chunks = rows_per_w // CR
    n_g = n_chunks // NBUF
    assert BATCH % NW == 0 and rows_per_w % CR == 0 and n_chunks % NBUF == 0
    splits = []
    s = 0
    while s < SEQ:
        w = min(IPG, SEQ - s)
        splits.append((s, w))
        s += w
    mesh = plsc.VectorSubcoreMesh(core_axis_name="c", subcore_axis_name="s")

    @functools.partial(
        pl.kernel,
        out_type=jax.ShapeDtypeStruct((BATCH, SEQ, D), jnp.float32),
        mesh=mesh,
        compiler_params=pltpu.CompilerParams(use_tc_tiling_on_sc=False),
        scratch_types=[
            pltpu.VMEM((NBUF, CR, SEQ), jnp.int32),
            pltpu.VMEM((NBUF, CR, SEQ, D), jnp.float32),
            pltpu.SemaphoreType.DMA((NBUF,)),
            pltpu.SemaphoreType.DMA((NBUF,)),
            pltpu.SemaphoreType.DMA,
        ],
    )
    def grab(idx_hbm, table_hbm, out_hbm, idx_v, rows_v, sem_i, sem_o, sem_g):
        wid = lax.axis_index("s") * NC + lax.axis_index("c")
        base = wid * rows_per_w

        def chunk_row(i):
            return pl.multiple_of(base + i * CR, CR)

        for b in range(NBUF):
            pltpu.async_copy(
                idx_hbm.at[pl.ds(chunk_row(b), CR)], idx_v.at[b], sem_i.at[b])

        def body(g, carry):
            for b in range(NBUF):
                i = g * NBUF + b

                @pl.when(g > 0)
                def _():
                    pltpu.make_async_copy(
                        rows_v.at[b],
                        out_hbm.at[pl.ds(chunk_row(i - NBUF), CR)],
                        sem_o.at[b],
                    ).wait()

                pltpu.make_async_copy(
                    idx_hbm.at[pl.ds(chunk_row(i), CR)], idx_v.at[b],
                    sem_i.at[b],
                ).wait()

                copies = [
                    pltpu.async_copy(
                        table_hbm.at[idx_v.at[b, r, pl.ds(s, w)]],
                        rows_v.at[b, r, pl.ds(s, w)],
                        sem_g,
                    )
                    for r in range(CR)
                    for (s, w) in splits
                ]
                for c in copies:
                    c.wait()

                @pl.when(g < n_g - 1)
                def _():
                    pltpu.async_copy(
                        idx_hbm.at[pl.ds(chunk_row(i + NBUF), CR)],
                        idx_v.at[b],
                        sem_i.at[b],
                    )

                pltpu.async_copy(
                    rows_v.at[b], out_hbm.at[pl.ds(chunk_row(i), CR)],
                    sem_o.at[b])
            return carry

        lax.fori_loop(0, n_g, body, 0)

        for b in range(NBUF):
            i = n_chunks - NBUF + b
            pltpu.make_async_copy(
                rows_v.at[b],
                out_hbm.at[pl.ds(chunk_row(i), CR)],
                sem_o.at[b],
            ).wait()

    return grab


def kernel(input, table):
    BATCH, SEQ = input.shape
    V, D = table.shape
    return _make_gather(BATCH, SEQ, V, D, 4, 2)(input, table)


# speedup vs baseline: 1.6989x; 1.6505x over previous
import functools

import jax
import jax.numpy as jnp
from jax import lax
from jax.experimental import pallas as pl
from jax.experimental.pallas import tpu as pltpu
from jax.experimental.pallas import tpu_sc as plsc

NC = 2
NS = 16
NW = NC * NS
IPG = 128


@functools.lru_cache(maxsize=None)
def _make_gather(BATCH, SEQ, V, D, CR, NBUF):
    rows_per_w = BATCH // NW
    n_chunks = rows_per_w // CR
    n_g = n_chunks // NBUF
    assert BATCH % NW == 0 and rows_per_w % CR == 0 and n_chunks % NBUF == 0
    splits = []
    s = 0
    while s < SEQ:
        w = min(IPG, SEQ - s)
        splits.append((s, w))
        s += w
    mesh = plsc.VectorSubcoreMesh(core_axis_name="c", subcore_axis_name="s")

    @functools.partial(
        pl.kernel,
        out_type=jax.ShapeDtypeStruct((BATCH, SEQ, 2 * D), jnp.float32),
        mesh=mesh,
        compiler_params=pltpu.CompilerParams(use_tc_tiling_on_sc=False),
        scratch_types=[
            pltpu.VMEM((NBUF, CR, SEQ), jnp.int32),
            pltpu.VMEM((NBUF, CR, SEQ, D), jnp.float32),
            pltpu.SemaphoreType.DMA((NBUF,)),
            pltpu.SemaphoreType.DMA((NBUF,)),
            pltpu.SemaphoreType.DMA,
        ],
    )
    def grab(idx_hbm, table_hbm, out_hbm, idx_v, rows_v, sem_i, sem_o, sem_g):
        wid = lax.axis_index("s") * NC + lax.axis_index("c")
        base = wid * rows_per_w

        def chunk_row(i):
            return pl.multiple_of(base + i * CR, CR)

        for b in range(NBUF):
            pltpu.async_copy(
                idx_hbm.at[pl.ds(chunk_row(b), CR)], idx_v.at[b], sem_i.at[b])

        def body(g, carry):
            for b in range(NBUF):
                i = g * NBUF + b

                @pl.when(g > 0)
                def _():
                    pltpu.make_async_copy(
                        rows_v.at[b],
                        out_hbm.at[pl.ds(chunk_row(i - NBUF), CR), :, pl.ds(0, D)],
                        sem_o.at[b],
                    ).wait()

                pltpu.make_async_copy(
                    idx_hbm.at[pl.ds(chunk_row(i), CR)], idx_v.at[b],
                    sem_i.at[b],
                ).wait()

                copies = [
                    pltpu.async_copy(
                        table_hbm.at[idx_v.at[b, r, pl.ds(s, w)]],
                        rows_v.at[b, r, pl.ds(s, w)],
                        sem_g,
                    )
                    for r in range(CR)
                    for (s, w) in splits
                ]
                for c in copies:
                    c.wait()

                @pl.when(g < n_g - 1)
                def _():
                    pltpu.async_copy(
                        idx_hbm.at[pl.ds(chunk_row(i + NBUF), CR)],
                        idx_v.at[b],
                        sem_i.at[b],
                    )

                pltpu.async_copy(
                    rows_v.at[b],
                    out_hbm.at[pl.ds(chunk_row(i), CR), :, pl.ds(0, D)],
                    sem_o.at[b])
            return carry

        lax.fori_loop(0, n_g, body, 0)

        for b in range(NBUF):
            i = n_chunks - NBUF + b
            pltpu.make_async_copy(
                rows_v.at[b],
                out_hbm.at[pl.ds(chunk_row(i), CR), :, pl.ds(0, D)],
                sem_o.at[b],
            ).wait()

    return grab


def kernel(input, table):
    BATCH, SEQ = input.shape
    V, D = table.shape
    out = _make_gather(BATCH, SEQ, V, D, 4, 2)(input, table)
    return out[:, :, :D]
